# baseline (speedup 1.0000x reference)
import functools

import jax
import jax.numpy as jnp
from jax import lax
from jax.experimental import pallas as pl
from jax.experimental.pallas import tpu as pltpu


_BT = 256
_NCLS = 10
_PCLS = 128
_CCH = 28
_PSP = 144
_CHK = 48


def _fused_kernel(xq_ref, wext_ref, bc_ref, wl_ref, bl_ref, o_ref):
    bt = o_ref.shape[0]
    xq = xq_ref[...]

    wins = []
    for ph in range(2):
        for pw in range(2):
            base = (ph * 2 + pw) * 14
            for bh in range(3):
                for bw in range(3):
                    wins.append(xq[base + bh: base + bh + 12, bw: bw + 12, :])
    pool = jnp.stack(wins, axis=0).reshape(36, _PSP, bt)

    wext = wext_ref[...]
    bcb = bc_ref[...].reshape(_CCH, 1, 1)
    chunks = []
    for k in range(_PSP // _CHK):
        ck = lax.dot_general(
            wext, pool[:, _CHK * k: _CHK * (k + 1), :],
            dimension_numbers=(((1,), (0,)), ((), ())),
            preferred_element_type=jnp.float32)
        pk = jnp.maximum(jnp.maximum(ck[0:28], ck[28:56]),
                         jnp.maximum(ck[56:84], ck[84:112]))
        chunks.append(jnp.maximum(pk + bcb, 0.0).astype(jnp.bfloat16))
    feats = jnp.stack(chunks, axis=0).reshape(_CCH * _PSP, bt)

    logits = lax.dot_general(
        feats, wl_ref[...],
        dimension_numbers=(((0,), (0,)), ((), ())),
        preferred_element_type=jnp.float32)
    logits = logits + bl_ref[...]
    m = jnp.max(logits, axis=-1, keepdims=True)
    s = logits - m
    lse = jnp.log(jnp.sum(jnp.exp(s), axis=-1, keepdims=True))
    o_ref[...] = s - lse


def _build_wext(conv_w):
    w = conv_w.reshape(_CCH, 5, 5)
    cols = []
    rows = []
    for a in range(2):
        for b in range(2):
            blk = jnp.zeros((_CCH, 36), conv_w.dtype)
            for kh in range(5):
                for kw in range(5):
                    v, wv = a + kh, b + kw
                    u = (v % 2) * 18 + (wv % 2) * 9 + (v // 2) * 3 + (wv // 2)
                    blk = blk.at[:, u].set(w[:, kh, kw])
            rows.append(blk)
    del cols
    return jnp.concatenate(rows, axis=0)


@functools.partial(jax.jit, static_argnames=())
def _forward(x, conv_w, conv_b, lin_w, lin_b):
    n = x.shape[0]
    bt = _BT
    n_pad = ((n + bt - 1) // bt) * bt
    x = x.astype(jnp.float32)
    if n_pad != n:
        x = jnp.pad(x, ((0, n_pad - n), (0, 0), (0, 0), (0, 0)))

    xs = x[:, 0].astype(jnp.bfloat16).reshape(n_pad, 14, 2, 14, 2)
    xq = jnp.transpose(xs, (2, 4, 1, 3, 0)).reshape(56, 14, n_pad)

    wext = _build_wext(conv_w).astype(jnp.bfloat16)
    bc = conv_b.reshape(_CCH, 1).astype(jnp.float32)
    wl3 = lin_w.astype(jnp.float32).reshape(_NCLS, _CCH, _PSP // _CHK, _CHK)
    wl3 = jnp.transpose(wl3, (2, 1, 3, 0)).reshape(_CCH * _PSP, _NCLS)
    wl = jnp.pad(wl3, ((0, 0), (0, _PCLS - _NCLS))).astype(jnp.bfloat16)
    bl = jnp.pad(lin_b.astype(jnp.float32), (0, _PCLS - _NCLS),
                 constant_values=-1e30).reshape(1, _PCLS)

    grid = (n_pad // bt,)
    out = pl.pallas_call(
        _fused_kernel,
        grid=grid,
        in_specs=[
            pl.BlockSpec((56, 14, bt), lambda i: (0, 0, i)),
            pl.BlockSpec((112, 36), lambda i: (0, 0)),
            pl.BlockSpec((_CCH, 1), lambda i: (0, 0)),
            pl.BlockSpec((_CCH * _PSP, _PCLS), lambda i: (0, 0)),
            pl.BlockSpec((1, _PCLS), lambda i: (0, 0)),
        ],
        out_specs=pl.BlockSpec((bt, _PCLS), lambda i: (i, 0)),
        out_shape=jax.ShapeDtypeStruct((n_pad, _PCLS), jnp.float32),
        compiler_params=pltpu.CompilerParams(
            dimension_semantics=("parallel",),
            vmem_limit_bytes=64 * 1024 * 1024),
    )(xq, wext, bc, wl, bl)

    return out[:n, :_NCLS]


def kernel(x, conv_w, conv_b, lin_w, lin_b):
    return _forward(x, conv_w, conv_b, lin_w, lin_b)


# speedup vs baseline: 1.0008x; 1.0008x over previous
import functools

import jax
import jax.numpy as jnp
from jax import lax
from jax.experimental import pallas as pl
from jax.experimental.pallas import tpu as pltpu


_BT = 256
_NCLS = 10
_PCLS = 128
_CCH = 28
_PSP = 144


def _fused_kernel(xq_ref, wext_ref, bc_ref, wl_ref, bl_ref, o_ref):
    bt = o_ref.shape[0]
    xq = xq_ref[...]

    wins = []
    for ph in range(2):
        for pw in range(2):
            base = (ph * 2 + pw) * 14
            for bh in range(3):
                for bw in range(3):
                    wins.append(xq[base + bh: base + bh + 12, bw: bw + 12, :])
    pool = jnp.stack(wins, axis=0).reshape(36, _PSP, bt)

    conv = lax.dot_general(
        wext_ref[...], pool,
        dimension_numbers=(((1,), (0,)), ((), ())),
        preferred_element_type=jnp.float32)

    pooled = jnp.maximum(jnp.maximum(conv[0:28], conv[28:56]),
                         jnp.maximum(conv[56:84], conv[84:112]))
    feats = jnp.maximum(pooled + bc_ref[...].reshape(_CCH, 1, 1), 0.0)
    feats = feats.astype(jnp.bfloat16).reshape(_CCH * _PSP, bt)

    logits = lax.dot_general(
        feats, wl_ref[...],
        dimension_numbers=(((0,), (0,)), ((), ())),
        preferred_element_type=jnp.float32)
    logits = logits + bl_ref[...]
    m = jnp.max(logits, axis=-1, keepdims=True)
    s = logits - m
    lse = jnp.log(jnp.sum(jnp.exp(s), axis=-1, keepdims=True))
    o_ref[...] = s - lse


def _build_wext(conv_w):
    w = conv_w.reshape(_CCH, 5, 5)
    rows = []
    for a in range(2):
        for b in range(2):
            blk = jnp.zeros((_CCH, 36), conv_w.dtype)
            for kh in range(5):
                for kw in range(5):
                    v, wv = a + kh, b + kw
                    u = (v % 2) * 18 + (wv % 2) * 9 + (v // 2) * 3 + (wv // 2)
                    blk = blk.at[:, u].set(w[:, kh, kw])
            rows.append(blk)
    return jnp.concatenate(rows, axis=0)


@functools.partial(jax.jit, static_argnames=())
def _forward(x, conv_w, conv_b, lin_w, lin_b):
    n = x.shape[0]
    bt = _BT
    n_pad = ((n + bt - 1) // bt) * bt
    x = x.astype(jnp.float32)
    if n_pad != n:
        x = jnp.pad(x, ((0, n_pad - n), (0, 0), (0, 0), (0, 0)))

    xs = x[:, 0].astype(jnp.bfloat16).reshape(n_pad, 14, 2, 14, 2)
    xq = jnp.transpose(xs, (2, 4, 1, 3, 0)).reshape(56, 14, n_pad)

    wext = _build_wext(conv_w).astype(jnp.bfloat16)
    bc = conv_b.reshape(_CCH, 1).astype(jnp.float32)
    wl = jnp.pad(lin_w.astype(jnp.float32).T,
                 ((0, 0), (0, _PCLS - _NCLS))).astype(jnp.bfloat16)
    bl = jnp.pad(lin_b.astype(jnp.float32), (0, _PCLS - _NCLS),
                 constant_values=-1e30).reshape(1, _PCLS)

    grid = (n_pad // bt,)
    out = pl.pallas_call(
        _fused_kernel,
        grid=grid,
        in_specs=[
            pl.BlockSpec((56, 14, bt), lambda i: (0, 0, i)),
            pl.BlockSpec((112, 36), lambda i: (0, 0)),
            pl.BlockSpec((_CCH, 1), lambda i: (0, 0)),
            pl.BlockSpec((_CCH * _PSP, _PCLS), lambda i: (0, 0)),
            pl.BlockSpec((1, _PCLS), lambda i: (0, 0)),
        ],
        out_specs=pl.BlockSpec((bt, _PCLS), lambda i: (i, 0)),
        out_shape=jax.ShapeDtypeStruct((n_pad, _PCLS), jnp.float32),
        compiler_params=pltpu.CompilerParams(
            dimension_semantics=("parallel",),
            vmem_limit_bytes=64 * 1024 * 1024),
    )(xq, wext, bc, wl, bl)

    return out[:n, :_NCLS]


def kernel(x, conv_w, conv_b, lin_w, lin_b):
    return _forward(x, conv_w, conv_b, lin_w, lin_b)
